# baseline (speedup 1.0000x reference)
import jax
import jax.numpy as jnp
from jax.experimental import pallas as pl
from jax.experimental.pallas import tpu as pltpu

BS_ = 2048
D_ = 768
NE_ = 8
EPS_ = 1


def _moe_kernel(x_ref, wg_ref, wn_ref, we_ref, be_ref, out_ref, xbf_ref, wt_ref):
    j = pl.program_id(0)

    @pl.when(j == 0)
    def _prologue():
        x = x_ref[...]
        xbf_ref[...] = (x * 4.0).astype(jnp.float8_e4m3fn)
        lg = jax.lax.dot_general(
            wg_ref[...], x, (((1,), (1,)), ((), ())),
            preferred_element_type=jnp.float32)
        ln = jax.lax.dot_general(
            wn_ref[...], x, (((1,), (1,)), ((), ())),
            preferred_element_type=jnp.float32)
        sp = jnp.maximum(ln, 0.0) + jnp.log1p(jnp.exp(-jnp.abs(ln)))
        logits = lg + sp
        m = jnp.max(logits, axis=0, keepdims=True)
        p = jnp.exp(logits - m)
        probs = p / jnp.sum(p, axis=0, keepdims=True)
        idx = jax.lax.broadcasted_iota(jnp.int32, (NE_, BS_), 0)
        m1 = jnp.max(probs, axis=0, keepdims=True)
        a1 = jnp.min(jnp.where(probs == m1, idx, NE_), axis=0, keepdims=True)
        masked = jnp.where(idx == a1, -jnp.inf, probs)
        m2 = jnp.max(masked, axis=0, keepdims=True)
        a2 = jnp.min(jnp.where(masked == m2, idx, NE_), axis=0, keepdims=True)
        i_lo = jnp.minimum(a1, a2)
        i_hi = jnp.maximum(a1, a2)
        w = (jnp.where(idx == i_lo, m1, 0.0)
             + jnp.where(idx == i_hi, m2, 0.0))
        wt_ref[...] = w * (1.0 / BS_)
        out_ref[...] = jnp.zeros_like(out_ref)

    xbf = xbf_ref[...]
    zs = []
    for r in range(EPS_):
        we_f8 = (we_ref[r] * 16.0).astype(jnp.float8_e4m3fn)
        zs.append(jax.lax.dot_general(
            xbf, we_f8, (((1,), (1,)), ((), ())),
            preferred_element_type=jnp.float32))
    acc = jnp.zeros((1, D_), dtype=jnp.float32)
    wsum = jnp.float32(0.0)
    for r in range(EPS_):
        e = j * EPS_ + r
        t = jnp.tanh(zs[r] * (0.5 / 64.0) + be_ref[r] * 0.5).astype(jnp.bfloat16)
        wrow = wt_ref[pl.ds(e, 1), :]
        acc += jax.lax.dot_general(
            wrow.astype(jnp.bfloat16), t, (((1,), (0,)), ((), ())),
            preferred_element_type=jnp.float32)
        wsum += jnp.sum(wrow)
    out_ref[...] += 0.5 * acc + 0.5 * wsum


def kernel(x, Wg, Wn, We, be):
    out = pl.pallas_call(
        _moe_kernel,
        grid=(NE_ // EPS_,),
        in_specs=[
            pl.BlockSpec((BS_, D_), lambda j: (0, 0)),
            pl.BlockSpec((NE_, D_), lambda j: (0, 0)),
            pl.BlockSpec((NE_, D_), lambda j: (0, 0)),
            pl.BlockSpec((EPS_, D_, D_), lambda j: (j, 0, 0)),
            pl.BlockSpec((EPS_, 1, D_), lambda j: (j, 0, 0)),
        ],
        out_specs=pl.BlockSpec((1, D_), lambda j: (0, 0)),
        out_shape=jax.ShapeDtypeStruct((1, D_), jnp.float32),
        scratch_shapes=[
            pltpu.VMEM((BS_, D_), jnp.float8_e4m3fn),
            pltpu.VMEM((NE_, BS_), jnp.float32),
        ],
    )(x, Wg, Wn, We, be.reshape(NE_, 1, D_))
    return out.reshape(D_)


# speedup vs baseline: 1.0567x; 1.0567x over previous
import jax
import jax.numpy as jnp
from jax.experimental import pallas as pl
from jax.experimental.pallas import tpu as pltpu

BS_ = 2048
D_ = 768
NE_ = 8
EPS_ = 2


def _moe_kernel(x_ref, wg_ref, wn_ref, we_ref, be_ref, out_ref, xbf_ref, wt_ref):
    j = pl.program_id(0)

    @pl.when(j == 0)
    def _prologue():
        x = x_ref[...]
        xbf_ref[...] = (x * 4.0).astype(jnp.float8_e4m3fn)
        lg = jax.lax.dot_general(
            wg_ref[...], x, (((1,), (1,)), ((), ())),
            preferred_element_type=jnp.float32)
        ln = jax.lax.dot_general(
            wn_ref[...], x, (((1,), (1,)), ((), ())),
            preferred_element_type=jnp.float32)
        sp = jnp.maximum(ln, 0.0) + jnp.log1p(jnp.exp(-jnp.abs(ln)))
        logits = lg + sp
        m = jnp.max(logits, axis=0, keepdims=True)
        p = jnp.exp(logits - m)
        probs = p / jnp.sum(p, axis=0, keepdims=True)
        idx = jax.lax.broadcasted_iota(jnp.int32, (NE_, BS_), 0)
        m1 = jnp.max(probs, axis=0, keepdims=True)
        a1 = jnp.min(jnp.where(probs == m1, idx, NE_), axis=0, keepdims=True)
        masked = jnp.where(idx == a1, -jnp.inf, probs)
        m2 = jnp.max(masked, axis=0, keepdims=True)
        a2 = jnp.min(jnp.where(masked == m2, idx, NE_), axis=0, keepdims=True)
        i_lo = jnp.minimum(a1, a2)
        i_hi = jnp.maximum(a1, a2)
        w = (jnp.where(idx == i_lo, m1, 0.0)
             + jnp.where(idx == i_hi, m2, 0.0))
        wt_ref[...] = w * (1.0 / BS_)
        out_ref[...] = jnp.zeros_like(out_ref)

    xbf = xbf_ref[...]
    zs = []
    for r in range(EPS_):
        we_f8 = (we_ref[r] * 16.0).astype(jnp.float8_e4m3fn)
        zs.append(jax.lax.dot_general(
            xbf, we_f8, (((1,), (1,)), ((), ())),
            preferred_element_type=jnp.float32))
    acc = jnp.zeros((1, D_), dtype=jnp.float32)
    wsum = jnp.float32(0.0)
    for r in range(EPS_):
        e = j * EPS_ + r
        t = jnp.tanh(zs[r] * (0.5 / 64.0) + be_ref[r] * 0.5).astype(jnp.bfloat16)
        wrow = wt_ref[pl.ds(e, 1), :]
        acc += jax.lax.dot_general(
            wrow.astype(jnp.bfloat16), t, (((1,), (0,)), ((), ())),
            preferred_element_type=jnp.float32)
        wsum += jnp.sum(wrow)
    out_ref[...] += 0.5 * acc + 0.5 * wsum


def kernel(x, Wg, Wn, We, be):
    out = pl.pallas_call(
        _moe_kernel,
        grid=(NE_ // EPS_,),
        in_specs=[
            pl.BlockSpec((BS_, D_), lambda j: (0, 0)),
            pl.BlockSpec((NE_, D_), lambda j: (0, 0)),
            pl.BlockSpec((NE_, D_), lambda j: (0, 0)),
            pl.BlockSpec((EPS_, D_, D_), lambda j: (j, 0, 0)),
            pl.BlockSpec((EPS_, 1, D_), lambda j: (j, 0, 0)),
        ],
        out_specs=pl.BlockSpec((1, D_), lambda j: (0, 0)),
        out_shape=jax.ShapeDtypeStruct((1, D_), jnp.float32),
        scratch_shapes=[
            pltpu.VMEM((BS_, D_), jnp.float8_e4m3fn),
            pltpu.VMEM((NE_, BS_), jnp.float32),
        ],
    )(x, Wg, Wn, We, be.reshape(NE_, 1, D_))
    return out.reshape(D_)
